# baseline (speedup 1.0000x reference)
import functools

import jax
import jax.numpy as jnp
from jax import lax
from jax.experimental import pallas as pl
from jax.experimental.pallas import tpu as pltpu
from jax.experimental.pallas import tpu_sc as plsc

NUM_RAYS = 65536
NS = 128
NB = NS + 1
NOUT = 2 * NB
NVREG = 16
HSTRIDE = 144
CSTRIDE = 256
NW = 32
RAYS_PER_W = NUM_RAYS // NW
CB = 16
NCHUNK = RAYS_PER_W // CB

HIST_PAD = 0.01
EPS = 1e-5

_mesh = plsc.VectorSubcoreMesh(
    core_axis_name="c", subcore_axis_name="s", num_cores=2, num_subcores=16
)


@functools.partial(
    pl.kernel,
    out_type=jax.ShapeDtypeStruct((NUM_RAYS, NOUT), jnp.float32),
    mesh=_mesh,
    compiler_params=pltpu.CompilerParams(
        needs_layout_passes=False, disable_bounds_checks=True),
    scratch_types=[
        pltpu.VMEM((CB, NS), jnp.float32),
        pltpu.VMEM((CB, NB), jnp.float32),
        pltpu.VMEM((CB * 144,), jnp.float32),
        pltpu.VMEM((CB,), jnp.float32),
        pltpu.VMEM((CB,), jnp.float32),
        pltpu.VMEM((CB * CSTRIDE,), jnp.float32),
        pltpu.VMEM((CB * HSTRIDE,), jnp.int32),
        pltpu.VMEM((CB * 264,), jnp.float32),
        pltpu.VMEM((CB, NOUT), jnp.float32),
    ],
)
def _sampler(w_hbm, eb_hbm, ne_hbm, fa_hbm, out_hbm,
             w_v, ebs_v, eb_v, ne_v, fa_v, cdf_v, h_v, out_f, out_v):
    wid = lax.axis_index("s") * 2 + lax.axis_index("c")
    base0 = wid * RAYS_PER_W

    iota = lax.broadcasted_iota(jnp.int32, (NVREG,), 0)
    ones_i = jnp.ones((NVREG,), jnp.int32)
    zeros_i = jnp.zeros((NVREG,), jnp.int32)
    zeros_f = jnp.zeros((NVREG,), jnp.float32)
    seed0 = jnp.where(iota == 0, 1, 0).astype(jnp.int32)
    zcell = iota * CSTRIDE + (CSTRIDE - 1)
    nv = NS // NVREG
    nh = HSTRIDE // NVREG
    ujc = [(iota + q * NVREG).astype(jnp.float32) * jnp.float32(1.0 / NB)
           + jnp.float32(1.0 / (2 * NB)) for q in range(nh)]

    def chunk_body(ci, _):
        base = base0 + ci * CB
        pltpu.sync_copy(w_hbm.at[pl.ds(base, CB)], w_v)
        pltpu.sync_copy(eb_hbm.at[pl.ds(base, CB)], ebs_v)
        pltpu.sync_copy(ne_hbm.at[pl.ds(base, CB)], ne_v)
        pltpu.sync_copy(fa_hbm.at[pl.ds(base, CB)], fa_v)
        plsc.store_scatter(cdf_v, [zcell], zeros_f)

        @plsc.parallel_loop(0, CB, 1, unroll=4)
        def ray_body(r):
            row = jnp.full((NVREG,), r, jnp.int32)
            h_sl = h_v.at[pl.ds(pl.multiple_of(r * HSTRIDE, 8), HSTRIDE)]
            cdf_sl = cdf_v.at[pl.ds(pl.multiple_of(r * CSTRIDE, 8), CSTRIDE)]
            eb_sl = eb_v.at[pl.ds(pl.multiple_of(r * 144, 8), 144)]
            out_sl = out_f.at[pl.ds(pl.multiple_of(r * 264, 8), 264)]
            for v in range(nv):
                eb_v[pl.ds(r * 144 + v * NVREG, NVREG)] = (
                    ebs_v[r, pl.ds(v * NVREG, NVREG)])
            eb_v[pl.ds(r * 144 + NB - NVREG, NVREG)] = (
                ebs_v[r, pl.ds(NB - NVREG, NVREG)])

            near = plsc.load_gather(ne_v, [row])
            fmn = plsc.load_gather(fa_v, [row]) - near

            h_v[pl.ds(r * HSTRIDE, NVREG)] = seed0
            for q in range(1, nh):
                h_v[pl.ds(r * HSTRIDE + q * NVREG, NVREG)] = zeros_i

            wv = [w_v[r, pl.ds(v * NVREG, NVREG)] + HIST_PAD for v in range(nv)]
            csr = [plsc.cumsum(wv[v]) for v in range(nv)]
            pre = [jnp.float32(0.0)]
            for v in range(nv):
                pre.append(pre[v] + csr[v][NVREG - 1])
            w_sum = pre[nv]
            pad = jnp.maximum(EPS - w_sum, 0.0)
            w_sum2 = w_sum + pad
            padc = pad * (1.0 / NS)
            recip = 1.0 / jnp.full((NVREG,), w_sum2, jnp.float32)
            kvec = (iota + 1).astype(jnp.float32)
            mcar = jnp.int32(0)
            for v in range(nv):
                cs = (csr[v] + pre[v] + (kvec + (v * NVREG)) * padc) * recip
                cdfv = jnp.minimum(cs, 1.0)
                cdf_sl[pl.ds(v * NVREG, NVREG)] = cdfv
                m = (cdfv * jnp.float32(NB) + 0.5).astype(jnp.int32)
                m = jnp.maximum(plsc.cummax(m), mcar)
                mcar = m[NVREG - 1]
                plsc.addupdate_scatter(h_sl, [m], ones_i)
                ebg = plsc.load_gather(eb_sl, [iota + (v * NVREG + 1)])
                eu_e = near + ebg * fmn
                pos_e = (iota + (v * NVREG + 1)) + m
                plsc.store_scatter(out_sl, [pos_e], eu_e)
            eb0 = plsc.load_gather(eb_sl, [zeros_i])
            plsc.store_scatter(out_sl, [zeros_i], near + eb0 * fmn,
                               mask=iota == 0)

            hv = [h_v[pl.ds(r * HSTRIDE + q * NVREG, NVREG)] for q in range(nh)]
            csq = [plsc.cumsum(hv[q]) for q in range(nh)]
            ipre = [jnp.int32(0)]
            for q in range(nh):
                ipre.append(ipre[q] + csq[q][NVREG - 1])
            for q in range(nh):
                inds = csq[q] + ipre[q]
                below = jnp.minimum(inds - 1, NS)
                above = jnp.minimum(inds, NS)
                cdf_below = plsc.load_gather(
                    cdf_sl, [(below - 1) & (CSTRIDE - 1)])
                cdf_above = plsc.load_gather(cdf_sl, [above - 1])
                eb_below = plsc.load_gather(eb_sl, [below])
                eb_above = plsc.load_gather(eb_sl, [above])
                denom = jnp.maximum(cdf_above - cdf_below, 1e-37)
                t = jnp.clip((ujc[q] - cdf_below) / denom, 0.0, 1.0)
                sample = eb_below + t * (eb_above - eb_below)
                eu_s = near + sample * fmn
                pos_s = inds + (iota + q * NVREG)
                if (q + 1) * NVREG <= NB:
                    plsc.store_scatter(out_sl, [pos_s], eu_s)
                else:
                    plsc.store_scatter(out_sl, [pos_s], eu_s,
                                       mask=(iota + q * NVREG) < NB)
            for q in range(NOUT // NVREG):
                out_v[r, pl.ds(q * NVREG, NVREG)] = (
                    out_f[pl.ds(r * 264 + q * NVREG, NVREG)])
            out_v[r, pl.ds(NOUT - NVREG, NVREG)] = (
                out_f[pl.ds(r * 264 + NOUT - NVREG, NVREG)])

        pltpu.sync_copy(out_v, out_hbm.at[pl.ds(base, CB)])
        return _

    lax.fori_loop(0, NCHUNK, chunk_body, 0)


def kernel(weights, existing_bins, nears, fars):
    w2 = weights[..., 0]
    ne = nears[:, 0]
    fa = fars[:, 0]
    return _sampler(w2, existing_bins, ne, fa)


# speedup vs baseline: 1.4958x; 1.4958x over previous
import functools

import jax
import jax.numpy as jnp
from jax import lax
from jax.experimental import pallas as pl
from jax.experimental.pallas import tpu as pltpu
from jax.experimental.pallas import tpu_sc as plsc

NUM_RAYS = 65536
NS = 128
NB = NS + 1
NOUT = 2 * NB
NVREG = 16
HSTRIDE = 144
CSTRIDE = 256
NW = 32
RAYS_PER_W = NUM_RAYS // NW
CB = 16
NCHUNK = RAYS_PER_W // CB

HIST_PAD = 0.01
EPS = 1e-5

_mesh = plsc.VectorSubcoreMesh(
    core_axis_name="c", subcore_axis_name="s", num_cores=2, num_subcores=16
)


@functools.partial(
    pl.kernel,
    out_type=jax.ShapeDtypeStruct((NUM_RAYS * NOUT,), jnp.float32),
    mesh=_mesh,
    compiler_params=pltpu.CompilerParams(
        needs_layout_passes=False, disable_bounds_checks=True),
    scratch_types=[
        pltpu.VMEM((CB, NS), jnp.float32),
        pltpu.VMEM((CB * NB,), jnp.float32),
        pltpu.VMEM((CB,), jnp.float32),
        pltpu.VMEM((CB,), jnp.float32),
        pltpu.VMEM((CB * CSTRIDE,), jnp.float32),
        pltpu.VMEM((CB * HSTRIDE,), jnp.int32),
        pltpu.VMEM((CB * NOUT,), jnp.float32),
    ],
)
def _sampler(w_hbm, eb_hbm, ne_hbm, fa_hbm, out_hbm,
             w_v, eb_v, ne_v, fa_v, cdf_v, h_v, out_v):
    wid = lax.axis_index("s") * 2 + lax.axis_index("c")
    base0 = wid * RAYS_PER_W

    iota = lax.broadcasted_iota(jnp.int32, (NVREG,), 0)
    ones_i = jnp.ones((NVREG,), jnp.int32)
    zeros_i = jnp.zeros((NVREG,), jnp.int32)
    zeros_f = jnp.zeros((NVREG,), jnp.float32)
    seed0 = jnp.where(iota == 0, 1, 0).astype(jnp.int32)
    zcell = iota * CSTRIDE + (CSTRIDE - 1)
    nv = NS // NVREG
    nh = HSTRIDE // NVREG
    ujc = [(iota + q * NVREG).astype(jnp.float32) * jnp.float32(1.0 / NB)
           + jnp.float32(1.0 / (2 * NB)) for q in range(nh)]

    def chunk_body(ci, _):
        base = base0 + ci * CB
        pltpu.sync_copy(w_hbm.at[pl.ds(base, CB)], w_v)
        pltpu.sync_copy(eb_hbm.at[pl.ds(base * NB, CB * NB)], eb_v)
        pltpu.sync_copy(ne_hbm.at[pl.ds(base, CB)], ne_v)
        pltpu.sync_copy(fa_hbm.at[pl.ds(base, CB)], fa_v)
        plsc.store_scatter(cdf_v, [zcell], zeros_f)

        @plsc.parallel_loop(0, CB, 1, unroll=4)
        def ray_body(r):
            row = jnp.full((NVREG,), r, jnp.int32)
            h_sl = h_v.at[pl.ds(pl.multiple_of(r * HSTRIDE, 8), HSTRIDE)]
            cdf_sl = cdf_v.at[pl.ds(pl.multiple_of(r * CSTRIDE, 8), CSTRIDE)]
            ebase = jnp.full((NVREG,), r * NB, jnp.int32)
            obase = jnp.full((NVREG,), r * NOUT, jnp.int32)

            near = plsc.load_gather(ne_v, [row])
            fmn = plsc.load_gather(fa_v, [row]) - near

            h_v[pl.ds(r * HSTRIDE, NVREG)] = seed0
            for q in range(1, nh):
                h_v[pl.ds(r * HSTRIDE + q * NVREG, NVREG)] = zeros_i

            wv = [w_v[r, pl.ds(v * NVREG, NVREG)] + HIST_PAD for v in range(nv)]
            csr = [plsc.cumsum(wv[v]) for v in range(nv)]
            pre = [jnp.float32(0.0)]
            for v in range(nv):
                pre.append(pre[v] + csr[v][NVREG - 1])
            w_sum = pre[nv]
            pad = jnp.maximum(EPS - w_sum, 0.0)
            w_sum2 = w_sum + pad
            padc = pad * (1.0 / NS)
            recip = 1.0 / jnp.full((NVREG,), w_sum2, jnp.float32)
            kvec = (iota + 1).astype(jnp.float32)
            mcar = jnp.int32(0)
            for v in range(nv):
                cs = (csr[v] + pre[v] + (kvec + (v * NVREG)) * padc) * recip
                cdfv = jnp.minimum(cs, 1.0)
                cdf_sl[pl.ds(v * NVREG, NVREG)] = cdfv
                m = (cdfv * jnp.float32(NB) + 0.5).astype(jnp.int32)
                m = jnp.maximum(plsc.cummax(m), mcar)
                mcar = m[NVREG - 1]
                plsc.addupdate_scatter(h_sl, [m], ones_i)
                ebg = plsc.load_gather(eb_v, [ebase + (iota + (v * NVREG + 1))])
                eu_e = near + ebg * fmn
                pos_e = (obase + (iota + (v * NVREG + 1))) + m
                plsc.store_scatter(out_v, [pos_e], eu_e)
            eb0 = plsc.load_gather(eb_v, [ebase])
            plsc.store_scatter(out_v, [obase], near + eb0 * fmn,
                               mask=iota == 0)

            hv = [h_v[pl.ds(r * HSTRIDE + q * NVREG, NVREG)] for q in range(nh)]
            csq = [plsc.cumsum(hv[q]) for q in range(nh)]
            ipre = [jnp.int32(0)]
            for q in range(nh):
                ipre.append(ipre[q] + csq[q][NVREG - 1])
            for q in range(nh):
                inds = csq[q] + ipre[q]
                below = jnp.minimum(inds - 1, NS)
                above = jnp.minimum(inds, NS)
                cdf_below = plsc.load_gather(
                    cdf_sl, [(below - 1) & (CSTRIDE - 1)])
                cdf_above = plsc.load_gather(cdf_sl, [above - 1])
                eb_below = plsc.load_gather(eb_v, [ebase + below])
                eb_above = plsc.load_gather(eb_v, [ebase + above])
                denom = jnp.maximum(cdf_above - cdf_below, 1e-37)
                t = jnp.clip((ujc[q] - cdf_below) / denom, 0.0, 1.0)
                sample = eb_below + t * (eb_above - eb_below)
                eu_s = near + sample * fmn
                pos_s = (obase + inds) + (iota + q * NVREG)
                if (q + 1) * NVREG <= NB:
                    plsc.store_scatter(out_v, [pos_s], eu_s)
                else:
                    plsc.store_scatter(out_v, [pos_s], eu_s,
                                       mask=(iota + q * NVREG) < NB)

        pltpu.sync_copy(out_v, out_hbm.at[pl.ds(base * NOUT, CB * NOUT)])
        return _

    lax.fori_loop(0, NCHUNK, chunk_body, 0)


def kernel(weights, existing_bins, nears, fars):
    w2 = weights[..., 0]
    eb1 = existing_bins.reshape(NUM_RAYS * NB)
    ne = nears[:, 0]
    fa = fars[:, 0]
    out = _sampler(w2, eb1, ne, fa)
    return out.reshape(NUM_RAYS, NOUT)


# speedup vs baseline: 1.6829x; 1.1250x over previous
import functools

import jax
import jax.numpy as jnp
from jax import lax
from jax.experimental import pallas as pl
from jax.experimental.pallas import tpu as pltpu
from jax.experimental.pallas import tpu_sc as plsc

NUM_RAYS = 65536
NS = 128
NB = NS + 1
NOUT = 2 * NB
NVREG = 16
HSTRIDE = 144
CSTRIDE = 256
NW = 32
RAYS_PER_W = NUM_RAYS // NW
CB = 64
NCHUNK = RAYS_PER_W // CB

HIST_PAD = 0.01
EPS = 1e-5

_mesh = plsc.VectorSubcoreMesh(
    core_axis_name="c", subcore_axis_name="s", num_cores=2, num_subcores=16
)


@functools.partial(
    pl.kernel,
    out_type=jax.ShapeDtypeStruct((NUM_RAYS * NOUT,), jnp.float32),
    mesh=_mesh,
    compiler_params=pltpu.CompilerParams(
        needs_layout_passes=False, disable_bounds_checks=True),
    scratch_types=[
        pltpu.VMEM((CB, NS), jnp.float32),
        pltpu.VMEM((CB * NB,), jnp.float32),
        pltpu.VMEM((CB,), jnp.float32),
        pltpu.VMEM((CB,), jnp.float32),
        pltpu.VMEM((CB * CSTRIDE,), jnp.float32),
        pltpu.VMEM((CB * HSTRIDE,), jnp.int32),
        pltpu.VMEM((CB * NOUT,), jnp.float32),
    ],
)
def _sampler(w_hbm, eb_hbm, ne_hbm, fa_hbm, out_hbm,
             w_v, eb_v, ne_v, fa_v, cdf_v, h_v, out_v):
    wid = lax.axis_index("s") * 2 + lax.axis_index("c")
    base0 = wid * RAYS_PER_W

    iota = lax.broadcasted_iota(jnp.int32, (NVREG,), 0)
    ones_i = jnp.ones((NVREG,), jnp.int32)
    zeros_i = jnp.zeros((NVREG,), jnp.int32)
    zeros_f = jnp.zeros((NVREG,), jnp.float32)
    seed0 = jnp.where(iota == 0, 1, 0).astype(jnp.int32)
    zcell = iota * CSTRIDE + (CSTRIDE - 1)
    nv = NS // NVREG
    nh = HSTRIDE // NVREG
    ujc = [(iota + q * NVREG).astype(jnp.float32) * jnp.float32(1.0 / NB)
           + jnp.float32(1.0 / (2 * NB)) for q in range(nh)]

    def chunk_body(ci, _):
        base = base0 + ci * CB
        pltpu.sync_copy(w_hbm.at[pl.ds(base, CB)], w_v)
        pltpu.sync_copy(eb_hbm.at[pl.ds(base * NB, CB * NB)], eb_v)
        pltpu.sync_copy(ne_hbm.at[pl.ds(base, CB)], ne_v)
        pltpu.sync_copy(fa_hbm.at[pl.ds(base, CB)], fa_v)
        for g in range(CB // NVREG):
            plsc.store_scatter(cdf_v, [zcell + g * (NVREG * CSTRIDE)], zeros_f)

        @plsc.parallel_loop(0, CB, 1, unroll=4)
        def ray_body(r):
            row = jnp.full((NVREG,), r, jnp.int32)
            h_sl = h_v.at[pl.ds(pl.multiple_of(r * HSTRIDE, 8), HSTRIDE)]
            cdf_sl = cdf_v.at[pl.ds(pl.multiple_of(r * CSTRIDE, 8), CSTRIDE)]
            ebase = jnp.full((NVREG,), r * NB, jnp.int32)
            obase = jnp.full((NVREG,), r * NOUT, jnp.int32)

            near = plsc.load_gather(ne_v, [row])
            fmn = plsc.load_gather(fa_v, [row]) - near

            h_v[pl.ds(r * HSTRIDE, NVREG)] = seed0
            for q in range(1, nh):
                h_v[pl.ds(r * HSTRIDE + q * NVREG, NVREG)] = zeros_i

            wv = [w_v[r, pl.ds(v * NVREG, NVREG)] + HIST_PAD for v in range(nv)]
            csr = [plsc.cumsum(wv[v]) for v in range(nv)]
            pre = [jnp.float32(0.0)]
            for v in range(nv):
                pre.append(pre[v] + csr[v][NVREG - 1])
            w_sum = pre[nv]
            pad = jnp.maximum(EPS - w_sum, 0.0)
            w_sum2 = w_sum + pad
            padc = pad * (1.0 / NS)
            recip = 1.0 / jnp.full((NVREG,), w_sum2, jnp.float32)
            kvec = (iota + 1).astype(jnp.float32)
            mcar = jnp.int32(0)
            for v in range(nv):
                cs = (csr[v] + pre[v] + (kvec + (v * NVREG)) * padc) * recip
                cdfv = jnp.minimum(cs, 1.0)
                cdf_sl[pl.ds(v * NVREG, NVREG)] = cdfv
                m = (cdfv * jnp.float32(NB) + 0.5).astype(jnp.int32)
                m = jnp.maximum(plsc.cummax(m), mcar)
                mcar = m[NVREG - 1]
                plsc.addupdate_scatter(h_sl, [m], ones_i)
                ebg = plsc.load_gather(eb_v, [ebase + (iota + (v * NVREG + 1))])
                eu_e = near + ebg * fmn
                pos_e = (obase + (iota + (v * NVREG + 1))) + m
                plsc.store_scatter(out_v, [pos_e], eu_e)
            eb0 = plsc.load_gather(eb_v, [ebase])
            plsc.store_scatter(out_v, [obase], near + eb0 * fmn,
                               mask=iota == 0)

            hv = [h_v[pl.ds(r * HSTRIDE + q * NVREG, NVREG)] for q in range(nh)]
            csq = [plsc.cumsum(hv[q]) for q in range(nh)]
            ipre = [jnp.int32(0)]
            for q in range(nh):
                ipre.append(ipre[q] + csq[q][NVREG - 1])
            for q in range(nh):
                inds = csq[q] + ipre[q]
                below = jnp.minimum(inds - 1, NS)
                above = jnp.minimum(inds, NS)
                cdf_below = plsc.load_gather(
                    cdf_sl, [(below - 1) & (CSTRIDE - 1)])
                cdf_above = plsc.load_gather(cdf_sl, [above - 1])
                eb_below = plsc.load_gather(eb_v, [ebase + below])
                eb_above = plsc.load_gather(eb_v, [ebase + above])
                denom = jnp.maximum(cdf_above - cdf_below, 1e-37)
                t = jnp.clip((ujc[q] - cdf_below) / denom, 0.0, 1.0)
                sample = eb_below + t * (eb_above - eb_below)
                eu_s = near + sample * fmn
                pos_s = (obase + inds) + (iota + q * NVREG)
                if (q + 1) * NVREG <= NB:
                    plsc.store_scatter(out_v, [pos_s], eu_s)
                else:
                    plsc.store_scatter(out_v, [pos_s], eu_s,
                                       mask=(iota + q * NVREG) < NB)

        pltpu.sync_copy(out_v, out_hbm.at[pl.ds(base * NOUT, CB * NOUT)])
        return _

    lax.fori_loop(0, NCHUNK, chunk_body, 0)


def kernel(weights, existing_bins, nears, fars):
    w2 = weights[..., 0]
    eb1 = existing_bins.reshape(NUM_RAYS * NB)
    ne = nears[:, 0]
    fa = fars[:, 0]
    out = _sampler(w2, eb1, ne, fa)
    return out.reshape(NUM_RAYS, NOUT)
